# Initial kernel scaffold; baseline (speedup 1.0000x reference)
import jax
import jax.numpy as jnp
from jax.experimental import pallas as pl


def kernel(edge_feats, node_feats, graph_feats, edge_feat_reflected, edge_index, idx_revs, W_jf, b_jf, W_if, b_if, W_ju, b_ju, W_iu, b_iu, W_ef, b_ef, W_ec, b_ec):
    raise NotImplementedError("write your pallas kernel here")


# speedup vs baseline: 13.7732x; 13.7732x over previous
import jax
import jax.numpy as jnp
from jax.experimental import pallas as pl

_NPER = 16
_EPER = _NPER * (_NPER - 1)
_ED = _NPER * _NPER
_A = 8
_MID = 8


def _mrf_block(ed_ref, er_ref, nf_ref, gf_ref, nd_ref,
               wjf_ref, bjf_ref, wju_ref, bju_ref,
               wif_ref, bif_ref, wiu_ref, biu_ref,
               wef_ref, bef_ref, dwec_ref, db_ref,
               out_ref):
    G = gf_ref.shape[0]
    EE = G * _ED
    NN = G * _NPER
    f32 = jnp.float32
    ed = ed_ref[...].reshape(EE, 64)
    er = er_ref[...].reshape(EE, 64)
    gf = gf_ref[...].reshape(G, 32)

    def edge_lin(w_ref, b_ref, cout):
        w = w_ref[...]
        u_part = jnp.dot(gf, w[64:, :], preferred_element_type=f32, precision=jax.lax.Precision.HIGHEST) + b_ref[...]
        x = jnp.dot(ed, w[:64, :], preferred_element_type=f32, precision=jax.lax.Precision.HIGHEST)
        xr = jnp.dot(er, w[:64, :], preferred_element_type=f32, precision=jax.lax.Precision.HIGHEST)
        x = (x.reshape(G, _ED, cout) + u_part[:, None, :]).reshape(EE, cout)
        xr = (xr.reshape(G, _ED, cout) + u_part[:, None, :]).reshape(EE, cout)
        return jnp.transpose(x), jnp.transpose(xr)

    fcT, fcrT = edge_lin(wjf_ref, bjf_ref, _A * _MID)
    ucT, ucrT = edge_lin(wju_ref, bju_ref, _A * _MID)
    efT, efrT = edge_lin(wef_ref, bef_ref, _MID)

    dwec = jnp.transpose(dwec_ref[...])
    d = jnp.sum(efT * efrT * dwec, axis=0, keepdims=True)
    d = d + db_ref[0, 0] + nd_ref[...].reshape(1, EE)
    eidx = jax.lax.broadcasted_iota(jnp.int32, (G, _ED), 1).reshape(1, EE)
    off_diag = (eidx % (_NPER + 1)) != 0
    flag = jnp.where(jnp.logical_and(d >= 0.0, off_diag), 1.0, 0.0)

    def node_lin(w_ref, b_ref):
        w = w_ref[...]
        nf = nf_ref[...].reshape(NN, 64)
        u_part = jnp.dot(gf, w[64:, :], preferred_element_type=f32, precision=jax.lax.Precision.HIGHEST) + b_ref[...]
        x = jnp.dot(nf, w[:64, :], preferred_element_type=f32, precision=jax.lax.Precision.HIGHEST)
        x = (x.reshape(G, _NPER, _A) + u_part[:, None, :]).reshape(NN, _A)
        return jnp.transpose(x)

    nifT = node_lin(wif_ref, bif_ref)
    niuT = node_lin(wiu_ref, biu_ref)

    fc3 = fcT.reshape(_A, _MID, EE)
    fcr3 = fcrT.reshape(_A, _MID, EE)
    uc3 = ucT.reshape(_A, _MID, EE)
    ucr3 = ucrT.reshape(_A, _MID, EE)

    def pair_vals(x3, y3):
        acc = x3[:, 0, :][:, None, :] * y3[:, 0, :][None, :, :]
        for k in range(1, _MID):
            acc = acc + x3[:, k, :][:, None, :] * y3[:, k, :][None, :, :]
        return acc

    fv = pair_vals(fc3, fcr3)
    uv = pair_vals(uc3, ucr3)
    fvT = jnp.transpose(fv, (1, 0, 2))
    uvT = jnp.transpose(uv, (1, 0, 2))

    ecol = jax.lax.broadcasted_iota(jnp.int32, (NN, EE), 1)
    nrow = jax.lax.broadcasted_iota(jnp.int32, (NN, EE), 0)
    same_g = (nrow // _NPER) == (ecol // _ED)
    n_of = nrow % _NPER
    i_of = (ecol % _ED) // _NPER
    j_of = ecol % _NPER
    Psrc = jnp.where(jnp.logical_and(same_g, n_of == i_of), 1.0, 0.0)
    Qdst = jnp.where(jnp.logical_and(same_g, n_of == j_of), 1.0, 0.0)
    SdstT = jnp.transpose(Qdst)

    def bcast(node_t, pmat):
        return jnp.dot(node_t, pmat, preferred_element_type=f32, precision=jax.lax.Precision.HIGHEST)

    if_s = bcast(nifT, Psrc)
    if_d = bcast(nifT, Qdst)
    iu_s = bcast(niuT, Psrc)
    iu_d = bcast(niuT, Qdst)

    def direction(mm_e, e_sub, if_e, nu_e, eu_sub, iu_e, FV, UV):
        agg = mm_e - e_sub + if_e
        m = FV + agg[None, :, :]
        mx = jnp.max(m, axis=1, keepdims=True)
        em = jnp.exp(m - mx)
        s = jnp.sum(em, axis=1)
        lse = mx[:, 0, :] + jnp.log(s)
        mx2 = jnp.max(lse, axis=0, keepdims=True)
        sh = lse - mx2
        m_sum = sh - jnp.log(jnp.sum(jnp.exp(sh), axis=0, keepdims=True))
        msg = m_sum * flag
        util = nu_e - eu_sub + iu_e
        tum = UV + util[None, :, :]
        util_msg = (jnp.sum(tum * em, axis=1) / s) * flag
        return msg, util_msg

    def body(_, st):
        mm, nu, e_msg, e_rmsg, e_util, e_rutil = st
        mm_s = bcast(mm, Psrc)
        mm_d = bcast(mm, Qdst)
        nu_s = bcast(nu, Psrc)
        nu_d = bcast(nu, Qdst)
        msg, util_msg = direction(mm_s, e_rmsg, if_s, nu_s, e_rutil, iu_s, fv, uv)
        msg_r, util_r = direction(mm_d, e_msg, if_d, nu_d, e_util, iu_d, fvT, uvT)
        new_mm = jnp.dot(msg, SdstT, preferred_element_type=f32, precision=jax.lax.Precision.HIGHEST)
        new_nu = jnp.dot(util_msg, SdstT, preferred_element_type=f32, precision=jax.lax.Precision.HIGHEST)
        return (new_mm, new_nu, msg, msg_r, util_msg, util_r)

    z_n = jnp.zeros((_A, NN), f32)
    z_e = jnp.zeros((_A, EE), f32)
    st = jax.lax.fori_loop(0, _NPER, body, (z_n, z_n, z_e, z_e, z_e, z_e))

    sel_r = jax.lax.broadcasted_iota(jnp.int32, (NN, G), 0)
    sel_c = jax.lax.broadcasted_iota(jnp.int32, (NN, G), 1)
    Rsel = jnp.where(sel_r == sel_c * _NPER, 1.0, 0.0)
    outT = jnp.dot(st[1] + niuT, Rsel, preferred_element_type=f32, precision=jax.lax.Precision.HIGHEST)
    out_ref[...] = jnp.transpose(outT).reshape(G, 1, _A)


def _densify(x, bg):
    c = x.shape[-1]
    y = x.reshape(bg, _NPER - 1, _NPER, c)
    y = jnp.pad(y, ((0, 0), (0, 0), (1, 0), (0, 0)))
    y = y.reshape(bg, (_NPER - 1) * (_NPER + 1), c)
    y = jnp.pad(y, ((0, 0), (0, 1), (0, 0)))
    return y


def kernel(edge_feats, node_feats, graph_feats, edge_feat_reflected, edge_index,
           idx_revs, W_jf, b_jf, W_if, b_if, W_ju, b_ju, W_iu, b_iu, W_ef, b_ef,
           W_ec, b_ec):
    E = edge_feats.shape[0]
    Bg = graph_feats.shape[0]

    u = jax.random.uniform(jax.random.key(1234), (E, 2), minval=1e-6,
                           maxval=1.0 - 1e-6, dtype=jnp.float32)
    g = -jnp.log(-jnp.log(u))
    canon = jnp.minimum(jnp.arange(E), idx_revs)
    gc = g[canon]
    nd = gc[:, 0] - gc[:, 1]

    ed = _densify(edge_feats, Bg)
    er = _densify(edge_feat_reflected, Bg)
    ndd = _densify(nd[:, None], Bg)[:, :, 0]
    nf3 = node_feats.reshape(Bg, _NPER, 64)

    b_jf2 = b_jf.reshape(1, -1)
    b_ju2 = b_ju.reshape(1, -1)
    b_if2 = b_if.reshape(1, -1)
    b_iu2 = b_iu.reshape(1, -1)
    b_ef2 = b_ef.reshape(1, -1)
    dwec = (W_ec[:, 0] - W_ec[:, 1]).reshape(1, _MID)
    db = (b_ec[0] - b_ec[1]).reshape(1, 1)

    G = 1
    grid = (Bg // G,)

    out = pl.pallas_call(
        _mrf_block,
        grid=grid,
        in_specs=[
            pl.BlockSpec((G, _ED, 64), lambda i: (i, 0, 0)),
            pl.BlockSpec((G, _ED, 64), lambda i: (i, 0, 0)),
            pl.BlockSpec((G, _NPER, 64), lambda i: (i, 0, 0)),
            pl.BlockSpec((G, 1, 32), lambda i: (i, 0, 0)),
            pl.BlockSpec((G, 1, _ED), lambda i: (i, 0, 0)),
            pl.BlockSpec((96, 64), lambda i: (0, 0)),
            pl.BlockSpec((1, 64), lambda i: (0, 0)),
            pl.BlockSpec((96, 64), lambda i: (0, 0)),
            pl.BlockSpec((1, 64), lambda i: (0, 0)),
            pl.BlockSpec((96, _A), lambda i: (0, 0)),
            pl.BlockSpec((1, _A), lambda i: (0, 0)),
            pl.BlockSpec((96, _A), lambda i: (0, 0)),
            pl.BlockSpec((1, _A), lambda i: (0, 0)),
            pl.BlockSpec((96, _MID), lambda i: (0, 0)),
            pl.BlockSpec((1, _MID), lambda i: (0, 0)),
            pl.BlockSpec((1, _MID), lambda i: (0, 0)),
            pl.BlockSpec((1, 1), lambda i: (0, 0)),
        ],
        out_specs=pl.BlockSpec((G, 1, _A), lambda i: (i, 0, 0)),
        out_shape=jax.ShapeDtypeStruct((Bg, 1, _A), jnp.float32),
    )(ed, er, nf3, graph_feats.reshape(Bg, 1, 32), ndd.reshape(Bg, 1, _ED),
      W_jf, b_jf2, W_ju, b_ju2, W_if, b_if2, W_iu, b_iu2, W_ef, b_ef2,
      dwec, db)
    return out.reshape(Bg, _A)


# speedup vs baseline: 16.7287x; 1.2146x over previous
import jax
import jax.numpy as jnp
from jax.experimental import pallas as pl

_NPER = 16
_EPER = _NPER * (_NPER - 1)
_ED = _NPER * _NPER
_A = 8
_MID = 8


def _mrf_block(ed_ref, er_ref, nf_ref, gf_ref, nd_ref,
               wjf_ref, bjf_ref, wju_ref, bju_ref,
               wif_ref, bif_ref, wiu_ref, biu_ref,
               wef_ref, bef_ref, dwec_ref, db_ref,
               out_ref):
    G = gf_ref.shape[0]
    EE = G * _ED
    NN = G * _NPER
    f32 = jnp.float32
    ed = ed_ref[...].reshape(EE, 64)
    er = er_ref[...].reshape(EE, 64)
    gf = gf_ref[...].reshape(G, 32)

    def edge_lin(w_ref, b_ref, cout):
        w = w_ref[...]
        u_part = jnp.dot(gf, w[64:, :], preferred_element_type=f32, precision=jax.lax.Precision.HIGHEST) + b_ref[...]
        x = jnp.dot(ed, w[:64, :], preferred_element_type=f32, precision=jax.lax.Precision.HIGHEST)
        xr = jnp.dot(er, w[:64, :], preferred_element_type=f32, precision=jax.lax.Precision.HIGHEST)
        x = (x.reshape(G, _ED, cout) + u_part[:, None, :]).reshape(EE, cout)
        xr = (xr.reshape(G, _ED, cout) + u_part[:, None, :]).reshape(EE, cout)
        return jnp.transpose(x), jnp.transpose(xr)

    fcT, fcrT = edge_lin(wjf_ref, bjf_ref, _A * _MID)
    ucT, ucrT = edge_lin(wju_ref, bju_ref, _A * _MID)
    efT, efrT = edge_lin(wef_ref, bef_ref, _MID)

    dwec = jnp.transpose(dwec_ref[...])
    d = jnp.sum(efT * efrT * dwec, axis=0, keepdims=True)
    d = d + db_ref[0, 0] + nd_ref[...].reshape(1, EE)
    eidx = jax.lax.broadcasted_iota(jnp.int32, (G, _ED), 1).reshape(1, EE)
    off_diag = (eidx % (_NPER + 1)) != 0
    flag = jnp.where(jnp.logical_and(d >= 0.0, off_diag), 1.0, 0.0)

    def node_lin(w_ref, b_ref):
        w = w_ref[...]
        nf = nf_ref[...].reshape(NN, 64)
        u_part = jnp.dot(gf, w[64:, :], preferred_element_type=f32, precision=jax.lax.Precision.HIGHEST) + b_ref[...]
        x = jnp.dot(nf, w[:64, :], preferred_element_type=f32, precision=jax.lax.Precision.HIGHEST)
        x = (x.reshape(G, _NPER, _A) + u_part[:, None, :]).reshape(NN, _A)
        return jnp.transpose(x)

    nifT = node_lin(wif_ref, bif_ref)
    niuT = node_lin(wiu_ref, biu_ref)

    fc3 = fcT.reshape(_A, _MID, EE)
    fcr3 = fcrT.reshape(_A, _MID, EE)
    uc3 = ucT.reshape(_A, _MID, EE)
    ucr3 = ucrT.reshape(_A, _MID, EE)

    def pair_vals(x3, y3):
        acc = x3[:, 0, :][:, None, :] * y3[:, 0, :][None, :, :]
        for k in range(1, _MID):
            acc = acc + x3[:, k, :][:, None, :] * y3[:, k, :][None, :, :]
        return acc

    fv = pair_vals(fc3, fcr3)
    uv = pair_vals(uc3, ucr3)
    fvT = jnp.transpose(fv, (1, 0, 2))
    uvT = jnp.transpose(uv, (1, 0, 2))

    ecol = jax.lax.broadcasted_iota(jnp.int32, (NN, EE), 1)
    nrow = jax.lax.broadcasted_iota(jnp.int32, (NN, EE), 0)
    same_g = (nrow // _NPER) == (ecol // _ED)
    n_of = nrow % _NPER
    i_of = (ecol % _ED) // _NPER
    j_of = ecol % _NPER
    Psrc = jnp.where(jnp.logical_and(same_g, n_of == i_of), 1.0, 0.0)
    Qdst = jnp.where(jnp.logical_and(same_g, n_of == j_of), 1.0, 0.0)
    SdstT = jnp.transpose(Qdst)

    def bcast(node_t, pmat):
        return jnp.dot(node_t, pmat, preferred_element_type=f32, precision=jax.lax.Precision.HIGHEST)

    if_s = bcast(nifT, Psrc)
    if_d = bcast(nifT, Qdst)
    iu_s = bcast(niuT, Psrc)
    iu_d = bcast(niuT, Qdst)

    def direction(mm_e, e_sub, if_e, nu_e, eu_sub, iu_e, FV, UV):
        agg = mm_e - e_sub + if_e
        m = FV + agg[None, :, :]
        mx = jnp.max(m, axis=1, keepdims=True)
        em = jnp.exp(m - mx)
        s = jnp.sum(em, axis=1)
        lse = mx[:, 0, :] + jnp.log(s)
        mx2 = jnp.max(lse, axis=0, keepdims=True)
        sh = lse - mx2
        m_sum = sh - jnp.log(jnp.sum(jnp.exp(sh), axis=0, keepdims=True))
        msg = m_sum * flag
        util = nu_e - eu_sub + iu_e
        tum = UV + util[None, :, :]
        util_msg = (jnp.sum(tum * em, axis=1) / s) * flag
        return msg, util_msg

    def body(_, st):
        mm, nu, e_msg, e_rmsg, e_util, e_rutil = st
        mm_s = bcast(mm, Psrc)
        mm_d = bcast(mm, Qdst)
        nu_s = bcast(nu, Psrc)
        nu_d = bcast(nu, Qdst)
        msg, util_msg = direction(mm_s, e_rmsg, if_s, nu_s, e_rutil, iu_s, fv, uv)
        msg_r, util_r = direction(mm_d, e_msg, if_d, nu_d, e_util, iu_d, fvT, uvT)
        new_mm = jnp.dot(msg, SdstT, preferred_element_type=f32, precision=jax.lax.Precision.HIGHEST)
        new_nu = jnp.dot(util_msg, SdstT, preferred_element_type=f32, precision=jax.lax.Precision.HIGHEST)
        return (new_mm, new_nu, msg, msg_r, util_msg, util_r)

    z_n = jnp.zeros((_A, NN), f32)
    z_e = jnp.zeros((_A, EE), f32)
    st = jax.lax.fori_loop(0, _NPER, body, (z_n, z_n, z_e, z_e, z_e, z_e))

    sel_r = jax.lax.broadcasted_iota(jnp.int32, (NN, G), 0)
    sel_c = jax.lax.broadcasted_iota(jnp.int32, (NN, G), 1)
    Rsel = jnp.where(sel_r == sel_c * _NPER, 1.0, 0.0)
    outT = jnp.dot(st[1] + niuT, Rsel, preferred_element_type=f32, precision=jax.lax.Precision.HIGHEST)
    out_ref[...] = jnp.transpose(outT).reshape(G, 1, _A)


def _densify(x, bg):
    c = x.shape[-1]
    y = x.reshape(bg, _NPER - 1, _NPER, c)
    y = jnp.pad(y, ((0, 0), (0, 0), (1, 0), (0, 0)))
    y = y.reshape(bg, (_NPER - 1) * (_NPER + 1), c)
    y = jnp.pad(y, ((0, 0), (0, 1), (0, 0)))
    return y


def kernel(edge_feats, node_feats, graph_feats, edge_feat_reflected, edge_index,
           idx_revs, W_jf, b_jf, W_if, b_if, W_ju, b_ju, W_iu, b_iu, W_ef, b_ef,
           W_ec, b_ec):
    E = edge_feats.shape[0]
    Bg = graph_feats.shape[0]

    u = jax.random.uniform(jax.random.key(1234), (E, 2), minval=1e-6,
                           maxval=1.0 - 1e-6, dtype=jnp.float32)
    g = -jnp.log(-jnp.log(u))
    canon = jnp.minimum(jnp.arange(E), idx_revs)
    gc = g[canon]
    nd = gc[:, 0] - gc[:, 1]

    ed = _densify(edge_feats, Bg)
    er = _densify(edge_feat_reflected, Bg)
    ndd = _densify(nd[:, None], Bg)[:, :, 0]
    nf3 = node_feats.reshape(Bg, _NPER, 64)

    b_jf2 = b_jf.reshape(1, -1)
    b_ju2 = b_ju.reshape(1, -1)
    b_if2 = b_if.reshape(1, -1)
    b_iu2 = b_iu.reshape(1, -1)
    b_ef2 = b_ef.reshape(1, -1)
    dwec = (W_ec[:, 0] - W_ec[:, 1]).reshape(1, _MID)
    db = (b_ec[0] - b_ec[1]).reshape(1, 1)

    G = 2
    grid = (Bg // G,)

    out = pl.pallas_call(
        _mrf_block,
        grid=grid,
        in_specs=[
            pl.BlockSpec((G, _ED, 64), lambda i: (i, 0, 0)),
            pl.BlockSpec((G, _ED, 64), lambda i: (i, 0, 0)),
            pl.BlockSpec((G, _NPER, 64), lambda i: (i, 0, 0)),
            pl.BlockSpec((G, 1, 32), lambda i: (i, 0, 0)),
            pl.BlockSpec((G, 1, _ED), lambda i: (i, 0, 0)),
            pl.BlockSpec((96, 64), lambda i: (0, 0)),
            pl.BlockSpec((1, 64), lambda i: (0, 0)),
            pl.BlockSpec((96, 64), lambda i: (0, 0)),
            pl.BlockSpec((1, 64), lambda i: (0, 0)),
            pl.BlockSpec((96, _A), lambda i: (0, 0)),
            pl.BlockSpec((1, _A), lambda i: (0, 0)),
            pl.BlockSpec((96, _A), lambda i: (0, 0)),
            pl.BlockSpec((1, _A), lambda i: (0, 0)),
            pl.BlockSpec((96, _MID), lambda i: (0, 0)),
            pl.BlockSpec((1, _MID), lambda i: (0, 0)),
            pl.BlockSpec((1, _MID), lambda i: (0, 0)),
            pl.BlockSpec((1, 1), lambda i: (0, 0)),
        ],
        out_specs=pl.BlockSpec((G, 1, _A), lambda i: (i, 0, 0)),
        out_shape=jax.ShapeDtypeStruct((Bg, 1, _A), jnp.float32),
    )(ed, er, nf3, graph_feats.reshape(Bg, 1, 32), ndd.reshape(Bg, 1, _ED),
      W_jf, b_jf2, W_ju, b_ju2, W_if, b_if2, W_iu, b_iu2, W_ef, b_ef2,
      dwec, db)
    return out.reshape(Bg, _A)


# speedup vs baseline: 17.1801x; 1.0270x over previous
import jax
import jax.numpy as jnp
from jax.experimental import pallas as pl

_NPER = 16
_EPER = _NPER * (_NPER - 1)
_ED = _NPER * _NPER
_A = 8
_MID = 8


def _mrf_block(ed_ref, er_ref, nf_ref, gf_ref, fl_ref,
               wjf_ref, bjf_ref, wju_ref, bju_ref,
               wif_ref, bif_ref, wiu_ref, biu_ref,
               out_ref):
    G = gf_ref.shape[0]
    EE = G * _ED
    NN = G * _NPER
    f32 = jnp.float32
    ed = ed_ref[...].reshape(EE, 64)
    er = er_ref[...].reshape(EE, 64)
    gf = gf_ref[...].reshape(G, 32)

    def edge_lin(w_ref, b_ref, cout):
        w = w_ref[...]
        u_part = jnp.dot(gf, w[64:, :], preferred_element_type=f32, precision=jax.lax.Precision.HIGHEST) + b_ref[...]
        x = jnp.dot(ed, w[:64, :], preferred_element_type=f32, precision=jax.lax.Precision.HIGHEST)
        xr = jnp.dot(er, w[:64, :], preferred_element_type=f32, precision=jax.lax.Precision.HIGHEST)
        x = (x.reshape(G, _ED, cout) + u_part[:, None, :]).reshape(EE, cout)
        xr = (xr.reshape(G, _ED, cout) + u_part[:, None, :]).reshape(EE, cout)
        return jnp.transpose(x), jnp.transpose(xr)

    fcT, fcrT = edge_lin(wjf_ref, bjf_ref, _A * _MID)
    ucT, ucrT = edge_lin(wju_ref, bju_ref, _A * _MID)

    flag = fl_ref[...].reshape(1, EE)

    def node_lin(w_ref, b_ref):
        w = w_ref[...]
        nf = nf_ref[...].reshape(NN, 64)
        u_part = jnp.dot(gf, w[64:, :], preferred_element_type=f32, precision=jax.lax.Precision.HIGHEST) + b_ref[...]
        x = jnp.dot(nf, w[:64, :], preferred_element_type=f32, precision=jax.lax.Precision.HIGHEST)
        x = (x.reshape(G, _NPER, _A) + u_part[:, None, :]).reshape(NN, _A)
        return jnp.transpose(x)

    nifT = node_lin(wif_ref, bif_ref)
    niuT = node_lin(wiu_ref, biu_ref)

    fc3 = fcT.reshape(_A, _MID, EE)
    fcr3 = fcrT.reshape(_A, _MID, EE)
    uc3 = ucT.reshape(_A, _MID, EE)
    ucr3 = ucrT.reshape(_A, _MID, EE)

    def pair_vals(x3, y3):
        acc = x3[:, 0, :][:, None, :] * y3[:, 0, :][None, :, :]
        for k in range(1, _MID):
            acc = acc + x3[:, k, :][:, None, :] * y3[:, k, :][None, :, :]
        return acc

    fv = pair_vals(fc3, fcr3)
    uv = pair_vals(uc3, ucr3)
    fvT = jnp.transpose(fv, (1, 0, 2))
    uvT = jnp.transpose(uv, (1, 0, 2))

    ecol = jax.lax.broadcasted_iota(jnp.int32, (NN, EE), 1)
    nrow = jax.lax.broadcasted_iota(jnp.int32, (NN, EE), 0)
    same_g = (nrow // _NPER) == (ecol // _ED)
    n_of = nrow % _NPER
    i_of = (ecol % _ED) // _NPER
    j_of = ecol % _NPER
    Psrc = jnp.where(jnp.logical_and(same_g, n_of == i_of), 1.0, 0.0)
    Qdst = jnp.where(jnp.logical_and(same_g, n_of == j_of), 1.0, 0.0)
    SdstT = jnp.transpose(Qdst)

    def bcast(node_t, pmat):
        return jnp.dot(node_t, pmat, preferred_element_type=f32, precision=jax.lax.Precision.HIGHEST)

    if_s = bcast(nifT, Psrc)
    if_d = bcast(nifT, Qdst)
    iu_s = bcast(niuT, Psrc)
    iu_d = bcast(niuT, Qdst)

    def direction(mm_e, e_sub, if_e, nu_e, eu_sub, iu_e, FV, UV):
        agg = mm_e - e_sub + if_e
        m = FV + agg[None, :, :]
        mx = jnp.max(m, axis=1, keepdims=True)
        em = jnp.exp(m - mx)
        s = jnp.sum(em, axis=1)
        lse = mx[:, 0, :] + jnp.log(s)
        mx2 = jnp.max(lse, axis=0, keepdims=True)
        sh = lse - mx2
        m_sum = sh - jnp.log(jnp.sum(jnp.exp(sh), axis=0, keepdims=True))
        msg = m_sum * flag
        util = nu_e - eu_sub + iu_e
        tum = UV + util[None, :, :]
        util_msg = (jnp.sum(tum * em, axis=1) / s) * flag
        return msg, util_msg

    def body(_, st):
        mm, nu, e_msg, e_rmsg, e_util, e_rutil = st
        mm_s = bcast(mm, Psrc)
        mm_d = bcast(mm, Qdst)
        nu_s = bcast(nu, Psrc)
        nu_d = bcast(nu, Qdst)
        msg, util_msg = direction(mm_s, e_rmsg, if_s, nu_s, e_rutil, iu_s, fv, uv)
        msg_r, util_r = direction(mm_d, e_msg, if_d, nu_d, e_util, iu_d, fvT, uvT)
        new_mm = jnp.dot(msg, SdstT, preferred_element_type=f32, precision=jax.lax.Precision.HIGHEST)
        new_nu = jnp.dot(util_msg, SdstT, preferred_element_type=f32, precision=jax.lax.Precision.HIGHEST)
        return (new_mm, new_nu, msg, msg_r, util_msg, util_r)

    z_n = jnp.zeros((_A, NN), f32)
    z_e = jnp.zeros((_A, EE), f32)
    st = jax.lax.fori_loop(0, _NPER, body, (z_n, z_n, z_e, z_e, z_e, z_e))

    sel_r = jax.lax.broadcasted_iota(jnp.int32, (NN, G), 0)
    sel_c = jax.lax.broadcasted_iota(jnp.int32, (NN, G), 1)
    Rsel = jnp.where(sel_r == sel_c * _NPER, 1.0, 0.0)
    outT = jnp.dot(st[1] + niuT, Rsel, preferred_element_type=f32, precision=jax.lax.Precision.HIGHEST)
    out_ref[...] = jnp.transpose(outT).reshape(G, 1, _A)


def _densify(x, bg):
    c = x.shape[-1]
    y = x.reshape(bg, _NPER - 1, _NPER, c)
    y = jnp.pad(y, ((0, 0), (0, 0), (1, 0), (0, 0)))
    y = y.reshape(bg, (_NPER - 1) * (_NPER + 1), c)
    y = jnp.pad(y, ((0, 0), (0, 1), (0, 0)))
    return y


def kernel(edge_feats, node_feats, graph_feats, edge_feat_reflected, edge_index,
           idx_revs, W_jf, b_jf, W_if, b_if, W_ju, b_ju, W_iu, b_iu, W_ef, b_ef,
           W_ec, b_ec):
    E = edge_feats.shape[0]
    Bg = graph_feats.shape[0]

    u_edge = jnp.repeat(graph_feats, E // Bg, axis=0)
    ein = jnp.concatenate([edge_feats, u_edge], axis=-1)
    einr = jnp.concatenate([edge_feat_reflected, u_edge], axis=-1)
    efr = ein @ W_ef + b_ef
    efr_r = einr @ W_ef + b_ef
    logits = (efr * efr_r) @ W_ec + b_ec
    u = jax.random.uniform(jax.random.key(1234), logits.shape, minval=1e-6,
                           maxval=1.0 - 1e-6, dtype=logits.dtype)
    g = -jnp.log(-jnp.log(u))
    canon = jnp.minimum(jnp.arange(E), idx_revs)
    g = g[canon]
    y_soft = jax.nn.softmax(logits + g, axis=-1)
    flag0 = jax.nn.one_hot(jnp.argmax(y_soft, axis=-1), 2,
                           dtype=y_soft.dtype)[:, 0]

    ed = _densify(edge_feats, Bg)
    er = _densify(edge_feat_reflected, Bg)
    fld = _densify(flag0[:, None], Bg)[:, :, 0]
    nf3 = node_feats.reshape(Bg, _NPER, 64)

    b_jf2 = b_jf.reshape(1, -1)
    b_ju2 = b_ju.reshape(1, -1)
    b_if2 = b_if.reshape(1, -1)
    b_iu2 = b_iu.reshape(1, -1)

    G = 2
    grid = (Bg // G,)

    out = pl.pallas_call(
        _mrf_block,
        grid=grid,
        in_specs=[
            pl.BlockSpec((G, _ED, 64), lambda i: (i, 0, 0)),
            pl.BlockSpec((G, _ED, 64), lambda i: (i, 0, 0)),
            pl.BlockSpec((G, _NPER, 64), lambda i: (i, 0, 0)),
            pl.BlockSpec((G, 1, 32), lambda i: (i, 0, 0)),
            pl.BlockSpec((G, 1, _ED), lambda i: (i, 0, 0)),
            pl.BlockSpec((96, 64), lambda i: (0, 0)),
            pl.BlockSpec((1, 64), lambda i: (0, 0)),
            pl.BlockSpec((96, 64), lambda i: (0, 0)),
            pl.BlockSpec((1, 64), lambda i: (0, 0)),
            pl.BlockSpec((96, _A), lambda i: (0, 0)),
            pl.BlockSpec((1, _A), lambda i: (0, 0)),
            pl.BlockSpec((96, _A), lambda i: (0, 0)),
            pl.BlockSpec((1, _A), lambda i: (0, 0)),
        ],
        out_specs=pl.BlockSpec((G, 1, _A), lambda i: (i, 0, 0)),
        out_shape=jax.ShapeDtypeStruct((Bg, 1, _A), jnp.float32),
    )(ed, er, nf3, graph_feats.reshape(Bg, 1, 32), fld.reshape(Bg, 1, _ED),
      W_jf, b_jf2, W_ju, b_ju2, W_if, b_if2, W_iu, b_iu2)
    return out.reshape(Bg, _A)
